# baseline (speedup 1.0000x reference)
import functools

import jax
import jax.numpy as jnp
from jax import lax
from jax.experimental import pallas as pl
from jax.experimental.pallas import tpu as pltpu
from jax.experimental.pallas import tpu_sc as plsc

N = 10000
NP = 10240
E = 320000
EP = 327680
D = 128
NC = 2
NS = 16
NW = NC * NS
EPW = EP // NW
CH = 64
NCH = EPW // CH
NBUF = 4
RPS = NP // NS
BLK = 2000
NBLK = N // BLK


def _make_sc_segment_sum(width, with_cnt, nbuf=NBUF):
  mesh = plsc.VectorSubcoreMesh(core_axis_name="c", subcore_axis_name="s")
  out_type = [jax.ShapeDtypeStruct((NC, NP, D), jnp.float32)]
  scratch = [
      pltpu.VMEM((nbuf, CH), jnp.int32),
      pltpu.VMEM((nbuf, CH), jnp.int32),
      pltpu.VMEM((nbuf, CH, width), jnp.float32),
      pltpu.VMEM_SHARED((NP, width), jnp.float32),
  ] + [pltpu.SemaphoreType.DMA] * (4 * nbuf)
  if with_cnt:
    out_type.append(jax.ShapeDtypeStruct((NC, NP, 16), jnp.float32))

  @functools.partial(
      pl.kernel,
      out_type=out_type,
      mesh=mesh,
      scratch_types=scratch,
      compiler_params=pltpu.CompilerParams(use_tc_tiling_on_sc=False),
  )
  def seg_sum(x_hbm, e_hbm, *rest):
    if with_cnt:
      feat_hbm, cnt_hbm, srcb, dstb, rows, acc = rest[:6]
      sems = rest[6:]
    else:
      feat_hbm, srcb, dstb, rows, acc = rest[:5]
      sems = rest[5:]
      cnt_hbm = None
    gsem = sems[0:nbuf]
    ssem = sems[nbuf:2 * nbuf]
    sfsem = sems[2 * nbuf:3 * nbuf]
    dfsem = sems[3 * nbuf:4 * nbuf]
    c = lax.axis_index("c")
    s = lax.axis_index("s")
    wid = s * NC + c
    cbase = wid * NCH

    @pl.loop(0, CH)
    def _(r):
      for k in range(width // 16):
        rows[0, r, pl.ds(k * 16, 16)] = jnp.zeros((16,), jnp.float32)

    for t in range(RPS // CH):
      pltpu.sync_copy(rows.at[0], acc.at[pl.ds(s * RPS + t * CH, CH)])
    plsc.subcore_barrier()

    def src_fetch(j, b):
      pltpu.async_copy(e_hbm.at[0, cbase + j], srcb.at[b], sfsem[b])

    def src_wait(j, b):
      pltpu.make_async_copy(e_hbm.at[0, cbase + j], srcb.at[b],
                            sfsem[b]).wait()

    def dst_fetch(j, b):
      pltpu.async_copy(e_hbm.at[1, cbase + j], dstb.at[b], dfsem[b])

    def dst_wait(j, b):
      pltpu.make_async_copy(e_hbm.at[1, cbase + j], dstb.at[b],
                            dfsem[b]).wait()

    def gather_start(b):
      pltpu.async_copy(x_hbm.at[srcb.at[b]], rows.at[b], gsem[b])

    def gather_wait(b):
      pltpu.make_async_copy(x_hbm.at[srcb.at[b]], rows.at[b],
                            gsem[b]).wait()

    def scatter_start(b):
      pltpu.async_copy(rows.at[b], acc.at[dstb.at[b]], ssem[b], add=True)

    def scatter_wait(b):
      pltpu.make_async_copy(rows.at[b], acc.at[dstb.at[b]],
                            ssem[b]).wait()

    ng = 4 if nbuf >= 5 else 3
    nsc = nbuf - ng
    for t in range(ng):
      pltpu.sync_copy(e_hbm.at[0, cbase + t], srcb.at[t])
    for t in range(2):
      pltpu.sync_copy(e_hbm.at[1, cbase + t], dstb.at[t])
    for t in range(ng):
      gather_start(t)
    src_fetch(ng, ng)
    for t in range(2, nbuf):
      dst_fetch(t, t)

    @pl.loop(0, NCH, step=nbuf)
    def _(j):
      for u in range(nbuf):
        jj = j + u
        b = u
        bg = (u + ng) % nbuf
        bs = (u + ng + 1) % nbuf
        gather_wait(b)

        @pl.when(jj >= 2)
        def _():
          dst_wait(jj, b)

        scatter_start(b)

        @pl.when(jj >= nsc)
        def _():
          scatter_wait(bg)
          @pl.when(jj < NCH - ng)
          def _():
            dst_fetch(jj + ng, bg)

        @pl.when(jj < NCH - ng)
        def _():
          src_wait(jj + ng, bg)
          gather_start(bg)

        @pl.when(jj < NCH - ng - 1)
        def _():
          src_fetch(jj + ng + 1, bs)

    for t in range(nsc):
      scatter_wait((NCH - nsc + t) % nbuf)
    plsc.subcore_barrier()
    if with_cnt:
      pltpu.sync_copy(acc.at[pl.ds(s * RPS, RPS), pl.ds(0, D)],
                      feat_hbm.at[c, pl.ds(s * RPS, RPS)])
      pltpu.sync_copy(acc.at[pl.ds(s * RPS, RPS), pl.ds(D, 16)],
                      cnt_hbm.at[c, pl.ds(s * RPS, RPS)])
    else:
      pltpu.sync_copy(acc.at[pl.ds(s * RPS, RPS)],
                      feat_hbm.at[c, pl.ds(s * RPS, RPS)])

  return seg_sum


_seg_sum_cnt = _make_sc_segment_sum(D + 16, with_cnt=True, nbuf=4)
_seg_sum_plain = _make_sc_segment_sum(D, with_cnt=False, nbuf=5)


def _tc_layer0(p, cntp, x, wl_t, wr_t, b):
  def body(p0_ref, p1_ref, c0_ref, c1_ref, x_ref, wl_ref, wr_ref, b_ref,
           h_ref, inv_ref):
    feat = p0_ref[0] + p1_ref[0]
    cnt = (c0_ref[0] + c1_ref[0])[:, 0:1]
    inv = 1.0 / jnp.maximum(cnt, 1.0)
    h = (jnp.dot(feat * inv, wl_ref[...], preferred_element_type=jnp.float32,
                 precision=lax.Precision.HIGHEST)
         + b_ref[...]
         + jnp.dot(x_ref[...], wr_ref[...], preferred_element_type=jnp.float32,
                   precision=lax.Precision.HIGHEST))
    h_ref[...] = jnp.maximum(h, 0.0)
    inv_ref[...] = jnp.broadcast_to(inv, (BLK, 8))

  return pl.pallas_call(
      body,
      grid=(NBLK,),
      in_specs=[
          pl.BlockSpec((1, BLK, D), lambda i: (0, i, 0)),
          pl.BlockSpec((1, BLK, D), lambda i: (1, i, 0)),
          pl.BlockSpec((1, BLK, 16), lambda i: (0, i, 0)),
          pl.BlockSpec((1, BLK, 16), lambda i: (1, i, 0)),
          pl.BlockSpec((BLK, D), lambda i: (i, 0)),
          pl.BlockSpec((D, D), lambda i: (0, 0)),
          pl.BlockSpec((D, D), lambda i: (0, 0)),
          pl.BlockSpec((1, D), lambda i: (0, 0)),
      ],
      out_specs=[
          pl.BlockSpec((BLK, D), lambda i: (i, 0)),
          pl.BlockSpec((BLK, 8), lambda i: (i, 0)),
      ],
      out_shape=[
          jax.ShapeDtypeStruct((N, D), jnp.float32),
          jax.ShapeDtypeStruct((N, 8), jnp.float32),
      ],
  )(p, p, cntp, cntp, x, wl_t, wr_t, b)


def _tc_layer1(q, h0, inv8, wl_t, wr_t, b):
  def body(q0_ref, q1_ref, h_ref, inv_ref, wl_ref, wr_ref, b_ref, o_ref):
    qa = q0_ref[0] + q1_ref[0]
    inv = inv_ref[...][:, 0:1]
    o = (jnp.dot(qa * inv, wl_ref[...], preferred_element_type=jnp.float32,
                 precision=lax.Precision.HIGHEST)
         + b_ref[...]
         + jnp.dot(h_ref[...], wr_ref[...], preferred_element_type=jnp.float32,
                   precision=lax.Precision.HIGHEST))
    o_ref[...] = jnp.maximum(o, 0.0)

  return pl.pallas_call(
      body,
      grid=(NBLK,),
      in_specs=[
          pl.BlockSpec((1, BLK, D), lambda i: (0, i, 0)),
          pl.BlockSpec((1, BLK, D), lambda i: (1, i, 0)),
          pl.BlockSpec((BLK, D), lambda i: (i, 0)),
          pl.BlockSpec((BLK, 8), lambda i: (i, 0)),
          pl.BlockSpec((D, D), lambda i: (0, 0)),
          pl.BlockSpec((D, D), lambda i: (0, 0)),
          pl.BlockSpec((1, D), lambda i: (0, 0)),
      ],
      out_specs=pl.BlockSpec((BLK, D), lambda i: (i, 0)),
      out_shape=jax.ShapeDtypeStruct((N, D), jnp.float32),
  )(q, q, h0, inv8, wl_t, wr_t, b)


def kernel(in_feat, edge_index, W0l, b0, W0r, W1l, b1, W1r):
  x_aug = jnp.concatenate(
      [in_feat,
       jnp.ones((N, 1), jnp.float32),
       jnp.zeros((N, 15), jnp.float32)], axis=1)
  pad_src = jnp.arange(EP - E, dtype=jnp.int32) % N
  pad_dst = N + (jnp.arange(EP - E, dtype=jnp.int32) % (NP - N))
  e_pad = jnp.concatenate(
      [edge_index, jnp.stack([pad_src, pad_dst])], axis=1)
  e_pad = e_pad.reshape(2, EP // CH, CH)
  p, cntp = _seg_sum_cnt(x_aug, e_pad)
  h0, inv8 = _tc_layer0(p, cntp, in_feat, W0l.T, W0r.T, b0.reshape(1, D))
  (q,) = _seg_sum_plain(h0, e_pad)
  return _tc_layer1(q, h0, inv8, W1l.T, W1r.T, b1.reshape(1, D))


# speedup vs baseline: 1.0019x; 1.0019x over previous
import functools

import jax
import jax.numpy as jnp
from jax import lax
from jax.experimental import pallas as pl
from jax.experimental.pallas import tpu as pltpu
from jax.experimental.pallas import tpu_sc as plsc

N = 10000
NP = 10240
E = 320000
EP = 327680
D = 128
NC = 2
NS = 16
NW = NC * NS
EPW = EP // NW
CH = 64
NCH = EPW // CH
NBUF = 4
RPS = NP // NS
BLK = 2000
NBLK = N // BLK


def _make_sc_segment_sum(width, with_cnt, nbuf=NBUF):
  mesh = plsc.VectorSubcoreMesh(core_axis_name="c", subcore_axis_name="s")
  out_type = [jax.ShapeDtypeStruct((NC, NP, D), jnp.float32)]
  scratch = [
      pltpu.VMEM((nbuf, CH), jnp.int32),
      pltpu.VMEM((nbuf, CH), jnp.int32),
      pltpu.VMEM((nbuf, CH, width), jnp.float32),
      pltpu.VMEM_SHARED((NP, width), jnp.float32),
  ] + [pltpu.SemaphoreType.DMA] * (4 * nbuf)
  if with_cnt:
    out_type.append(jax.ShapeDtypeStruct((NC, NP, 16), jnp.float32))

  @functools.partial(
      pl.kernel,
      out_type=out_type,
      mesh=mesh,
      scratch_types=scratch,
      compiler_params=pltpu.CompilerParams(use_tc_tiling_on_sc=False),
  )
  def seg_sum(x_hbm, e_hbm, *rest):
    if with_cnt:
      feat_hbm, cnt_hbm, srcb, dstb, rows, acc = rest[:6]
      sems = rest[6:]
    else:
      feat_hbm, srcb, dstb, rows, acc = rest[:5]
      sems = rest[5:]
      cnt_hbm = None
    gsem = sems[0:nbuf]
    ssem = sems[nbuf:2 * nbuf]
    sfsem = sems[2 * nbuf:3 * nbuf]
    dfsem = sems[3 * nbuf:4 * nbuf]
    c = lax.axis_index("c")
    s = lax.axis_index("s")
    wid = s * NC + c
    cbase = wid * NCH

    @pl.loop(0, CH)
    def _(r):
      for k in range(width // 16):
        rows[0, r, pl.ds(k * 16, 16)] = jnp.zeros((16,), jnp.float32)

    for t in range(RPS // CH):
      pltpu.sync_copy(rows.at[0], acc.at[pl.ds(s * RPS + t * CH, CH)])
    plsc.subcore_barrier()

    def src_fetch(j, b):
      pltpu.async_copy(e_hbm.at[0, cbase + j], srcb.at[b], sfsem[b])

    def src_wait(j, b):
      pltpu.make_async_copy(e_hbm.at[0, cbase + j], srcb.at[b],
                            sfsem[b]).wait()

    def dst_fetch(j, b):
      pltpu.async_copy(e_hbm.at[1, cbase + j], dstb.at[b], dfsem[b])

    def dst_wait(j, b):
      pltpu.make_async_copy(e_hbm.at[1, cbase + j], dstb.at[b],
                            dfsem[b]).wait()

    def gather_start(b):
      pltpu.async_copy(x_hbm.at[srcb.at[b]], rows.at[b], gsem[b])

    def gather_wait(b):
      pltpu.make_async_copy(x_hbm.at[srcb.at[b]], rows.at[b],
                            gsem[b]).wait()

    def scatter_start(b):
      pltpu.async_copy(rows.at[b], acc.at[dstb.at[b]], ssem[b], add=True)

    def scatter_wait(b):
      pltpu.make_async_copy(rows.at[b], acc.at[dstb.at[b]],
                            ssem[b]).wait()

    ng = 3
    nsc = nbuf - ng
    for t in range(ng):
      pltpu.sync_copy(e_hbm.at[0, cbase + t], srcb.at[t])
    for t in range(2):
      pltpu.sync_copy(e_hbm.at[1, cbase + t], dstb.at[t])
    for t in range(ng):
      gather_start(t)
    src_fetch(ng, ng)
    for t in range(2, nbuf):
      dst_fetch(t, t)

    @pl.loop(0, NCH, step=nbuf)
    def _(j):
      for u in range(nbuf):
        jj = j + u
        b = u
        bg = (u + ng) % nbuf
        bs = (u + ng + 1) % nbuf
        gather_wait(b)

        @pl.when(jj >= 2)
        def _():
          dst_wait(jj, b)

        scatter_start(b)

        @pl.when(jj >= nsc)
        def _():
          scatter_wait(bg)
          @pl.when(jj < NCH - ng)
          def _():
            dst_fetch(jj + ng, bg)

        @pl.when(jj < NCH - ng)
        def _():
          src_wait(jj + ng, bg)
          gather_start(bg)

        @pl.when(jj < NCH - ng - 1)
        def _():
          src_fetch(jj + ng + 1, bs)

    for t in range(nsc):
      scatter_wait((NCH - nsc + t) % nbuf)
    plsc.subcore_barrier()
    if with_cnt:
      pltpu.sync_copy(acc.at[pl.ds(s * RPS, RPS), pl.ds(0, D)],
                      feat_hbm.at[c, pl.ds(s * RPS, RPS)])
      pltpu.sync_copy(acc.at[pl.ds(s * RPS, RPS), pl.ds(D, 16)],
                      cnt_hbm.at[c, pl.ds(s * RPS, RPS)])
    else:
      pltpu.sync_copy(acc.at[pl.ds(s * RPS, RPS)],
                      feat_hbm.at[c, pl.ds(s * RPS, RPS)])

  return seg_sum


_seg_sum_cnt = _make_sc_segment_sum(D + 16, with_cnt=True, nbuf=4)
_seg_sum_plain = _make_sc_segment_sum(D, with_cnt=False, nbuf=5)


def _tc_layer0(p, cntp, x, wl_t, wr_t, b):
  def body(p0_ref, p1_ref, c0_ref, c1_ref, x_ref, wl_ref, wr_ref, b_ref,
           h_ref, inv_ref):
    feat = p0_ref[0] + p1_ref[0]
    cnt = (c0_ref[0] + c1_ref[0])[:, 0:1]
    inv = 1.0 / jnp.maximum(cnt, 1.0)
    h = (jnp.dot(feat * inv, wl_ref[...], preferred_element_type=jnp.float32,
                 precision=lax.Precision.HIGHEST)
         + b_ref[...]
         + jnp.dot(x_ref[...], wr_ref[...], preferred_element_type=jnp.float32,
                   precision=lax.Precision.HIGHEST))
    h_ref[...] = jnp.maximum(h, 0.0)
    inv_ref[...] = jnp.broadcast_to(inv, (BLK, 8))

  return pl.pallas_call(
      body,
      grid=(NBLK,),
      in_specs=[
          pl.BlockSpec((1, BLK, D), lambda i: (0, i, 0)),
          pl.BlockSpec((1, BLK, D), lambda i: (1, i, 0)),
          pl.BlockSpec((1, BLK, 16), lambda i: (0, i, 0)),
          pl.BlockSpec((1, BLK, 16), lambda i: (1, i, 0)),
          pl.BlockSpec((BLK, D), lambda i: (i, 0)),
          pl.BlockSpec((D, D), lambda i: (0, 0)),
          pl.BlockSpec((D, D), lambda i: (0, 0)),
          pl.BlockSpec((1, D), lambda i: (0, 0)),
      ],
      out_specs=[
          pl.BlockSpec((BLK, D), lambda i: (i, 0)),
          pl.BlockSpec((BLK, 8), lambda i: (i, 0)),
      ],
      out_shape=[
          jax.ShapeDtypeStruct((N, D), jnp.float32),
          jax.ShapeDtypeStruct((N, 8), jnp.float32),
      ],
  )(p, p, cntp, cntp, x, wl_t, wr_t, b)


def _tc_layer1(q, h0, inv8, wl_t, wr_t, b):
  def body(q0_ref, q1_ref, h_ref, inv_ref, wl_ref, wr_ref, b_ref, o_ref):
    qa = q0_ref[0] + q1_ref[0]
    inv = inv_ref[...][:, 0:1]
    o = (jnp.dot(qa * inv, wl_ref[...], preferred_element_type=jnp.float32,
                 precision=lax.Precision.HIGHEST)
         + b_ref[...]
         + jnp.dot(h_ref[...], wr_ref[...], preferred_element_type=jnp.float32,
                   precision=lax.Precision.HIGHEST))
    o_ref[...] = jnp.maximum(o, 0.0)

  return pl.pallas_call(
      body,
      grid=(NBLK,),
      in_specs=[
          pl.BlockSpec((1, BLK, D), lambda i: (0, i, 0)),
          pl.BlockSpec((1, BLK, D), lambda i: (1, i, 0)),
          pl.BlockSpec((BLK, D), lambda i: (i, 0)),
          pl.BlockSpec((BLK, 8), lambda i: (i, 0)),
          pl.BlockSpec((D, D), lambda i: (0, 0)),
          pl.BlockSpec((D, D), lambda i: (0, 0)),
          pl.BlockSpec((1, D), lambda i: (0, 0)),
      ],
      out_specs=pl.BlockSpec((BLK, D), lambda i: (i, 0)),
      out_shape=jax.ShapeDtypeStruct((N, D), jnp.float32),
  )(q, q, h0, inv8, wl_t, wr_t, b)


def kernel(in_feat, edge_index, W0l, b0, W0r, W1l, b1, W1r):
  x_aug = jnp.concatenate(
      [in_feat,
       jnp.ones((N, 1), jnp.float32),
       jnp.zeros((N, 15), jnp.float32)], axis=1)
  pad_src = jnp.arange(EP - E, dtype=jnp.int32) % N
  pad_dst = N + (jnp.arange(EP - E, dtype=jnp.int32) % (NP - N))
  e_pad = jnp.concatenate(
      [edge_index, jnp.stack([pad_src, pad_dst])], axis=1)
  e_pad = e_pad.reshape(2, EP // CH, CH)
  p, cntp = _seg_sum_cnt(x_aug, e_pad)
  h0, inv8 = _tc_layer0(p, cntp, in_feat, W0l.T, W0r.T, b0.reshape(1, D))
  (q,) = _seg_sum_plain(h0, e_pad)
  return _tc_layer1(q, h0, inv8, W1l.T, W1r.T, b1.reshape(1, D))
